# baseline (speedup 1.0000x reference)
import functools

import jax
import jax.numpy as jnp
from jax import lax
from jax.experimental import pallas as pl
from jax.experimental.pallas import tpu as pltpu
from jax.experimental.pallas import tpu_sc as plsc

_NC = 2
_NS = 16
_L = 16
_NW = _NC * _NS

_MULT_CUR = 36313
_MULT_PREV = 27191

_SLAB = 512
_N_SLABS = 64
_BCAP = 32
_GRP = 1024
_CHUNK_T = 1024


def _make_sc_kernel(N, V, D):
    vtail = (V // 128) * 128
    max_off = ((V - _SLAB) // _SLAB) * _SLAB
    per_tile = V // _NW
    mesh = plsc.VectorSubcoreMesh(core_axis_name="c", subcore_axis_name="s")

    @functools.partial(
        pl.kernel,
        out_type=jax.ShapeDtypeStruct((N, 2 * D), jnp.float32),
        mesh=mesh,
        scratch_types=[
            pltpu.VMEM((_CHUNK_T,), jnp.int32),
            pltpu.VMEM((_CHUNK_T,), jnp.int32),
            pltpu.VMEM((N,), jnp.int32),
            pltpu.VMEM((N,), jnp.int32),
            pltpu.VMEM((65 * _BCAP,), jnp.int32),
            pltpu.VMEM((65 * _BCAP,), jnp.int32),
            pltpu.VMEM((64, _SLAB), jnp.float32),
            pltpu.VMEM((64, _SLAB), jnp.float32),
            pltpu.VMEM((64, 64), jnp.float32),
            pltpu.VMEM((128, 128), jnp.float32),
            pltpu.VMEM((64,), jnp.int32),
            pltpu.VMEM((64,), jnp.int32),
            pltpu.VMEM((_L,), jnp.int32),
            pltpu.VMEM((_L,), jnp.int32),
            pltpu.VMEM((_L,), jnp.float32),
            pltpu.SMEM((66,), jnp.int32),
            pltpu.SemaphoreType.DMA,
            pltpu.SemaphoreType.DMA,
            pltpu.SemaphoreType.DMA,
            pltpu.SemaphoreType.DMA,
        ],
        compiler_params=pltpu.CompilerParams(needs_layout_passes=False),
    )
    def sc_kernel(t_hbm, p_hbm, s_hbm, tail_hbm, xt_hbm, out_hbm,
                  t_v, p_v, mh_v, mi_v, bh_v, bi_v, slab0_v, slab1_v,
                  tail_v, sw_v, sia_v, sib_v, sth_v, sti_v, s_v, cnts,
                  sem_in, sem_s0, sem_s1, sem_sc):
        wid = lax.axis_index("s") * _NC + lax.axis_index("c")
        lo = wid * per_tile
        hi = lo + per_tile
        hi_dma = jnp.minimum(hi, vtail)
        lo_al = (lo // _SLAB) * _SLAB
        iota = lax.iota(jnp.int32, _L)
        lane0 = iota == 0

        def slab_off(k):
            return pl.multiple_of(
                jnp.minimum(lo_al + k * _SLAB, max_off), _SLAB)

        def fire_slab(k, buf, sem):
            pltpu.async_copy(
                xt_hbm.at[:, pl.ds(slab_off(k), _SLAB)], buf, sem)

        def wait_slab(buf, sem):
            pltpu.make_async_copy(
                xt_hbm.at[:, pl.ds(0, _SLAB)], buf, sem).wait()

        fire_slab(0, slab0_v, sem_s0)
        fire_slab(1, slab1_v, sem_s1)
        pltpu.sync_copy(s_hbm, s_v)
        pltpu.async_copy(tail_hbm, tail_v, sem_in).wait()
        sv = s_v[...]

        for j in range(66):
            cnts[j] = jnp.int32(0)
        neg1 = jnp.full((_L,), -1, jnp.int32)
        for j in range(64 // _L):
            sia_v[pl.ds(j * _L, _L)] = neg1
            sib_v[pl.ds(j * _L, _L)] = neg1

        cnt = jnp.int32(0)
        for cc in range(N // _CHUNK_T):
            cp_t = pltpu.async_copy(
                t_hbm.at[pl.ds(cc * _CHUNK_T, _CHUNK_T)], t_v, sem_in)
            cp_p = pltpu.async_copy(
                p_hbm.at[pl.ds(cc * _CHUNK_T, _CHUNK_T)], p_v, sem_in)
            cp_t.wait()
            cp_p.wait()

            def hash_vreg(j, cnt, cc=cc):
                cur = t_v[pl.ds(j * _L, _L)]
                prv = p_v[pl.ds(j * _L, _L)]
                h = lax.bitwise_xor(
                    cur * _MULT_CUR, prv * _MULT_PREV) % (V - 1)
                m = (h >= lo) & (h < hi)
                pc = plsc.all_reduce_population_count(m)[0]

                @pl.when(pc > 0)
                def _():
                    pos = cnt + plsc.cumsum(m.astype(jnp.int32)) - 1
                    plsc.store_scatter(mh_v, [pos], h, mask=m)
                    plsc.store_scatter(
                        mi_v, [pos], cc * _CHUNK_T + j * _L + iota, mask=m)

                return cnt + pc

            cnt = lax.fori_loop(0, _CHUNK_T // _L, hash_vreg, cnt)

        n_mv = (cnt + _L - 1) // _L

        def bucket_vreg(q, _):
            @pl.when(q * _L < cnt)
            def _():
                hv = mh_v[pl.ds(q * _L, _L)]
                iv = mi_v[pl.ds(q * _L, _L)]
                kv = jnp.where(hv >= vtail, 64, (hv - lo_al) >> 9)
                for ln in range(_L):
                    @pl.when(q * _L + ln < cnt)
                    def _(ln=ln):
                        k_s = kv[ln]
                        c_s = cnts[k_s]
                        cnts[k_s] = c_s + 1

                        @pl.when(c_s < _BCAP)
                        def _():
                            p16 = jnp.zeros((_L,), jnp.int32) + (
                                k_s * _BCAP + c_s)
                            plsc.store_scatter(
                                bh_v, [p16],
                                jnp.zeros((_L,), jnp.int32) + hv[ln],
                                mask=lane0)
                            plsc.store_scatter(
                                bi_v, [p16],
                                jnp.zeros((_L,), jnp.int32) + iv[ln],
                                mask=lane0)
            return 0

        lax.fori_loop(0, N // _L, bucket_vreg, 0)

        def flush_wait(parity):
            @pl.when(parity == 0)
            def _():
                pltpu.make_async_copy(
                    sw_v.at[pl.ds(0, 64), :],
                    out_hbm.at[plsc.Indices(sia_v, ignored_value=-1)],
                    sem_sc).wait()

            @pl.when(parity == 1)
            def _():
                pltpu.make_async_copy(
                    sw_v.at[pl.ds(64, 64), :],
                    out_hbm.at[plsc.Indices(sib_v, ignored_value=-1)],
                    sem_sc).wait()

        def flush_fire(parity):
            @pl.when(parity == 0)
            def _():
                pltpu.async_copy(
                    sw_v.at[pl.ds(0, 64), :],
                    out_hbm.at[plsc.Indices(sia_v, ignored_value=-1)],
                    sem_sc)

            @pl.when(parity == 1)
            def _():
                pltpu.async_copy(
                    sw_v.at[pl.ds(64, 64), :],
                    out_hbm.at[plsc.Indices(sib_v, ignored_value=-1)],
                    sem_sc)

        def emit_group(src_v, col_base, h16, i16, nvalid):
            gc = cnts[65]
            slot = gc % 4
            parity = (gc // 4) % 2

            @pl.when((slot == 0) & (gc >= 8))
            def _():
                flush_wait(parity)

            valid = iota < nvalid
            col = h16 - col_base
            row16 = parity * 64 + slot * _L + iota
            siv = jnp.where(valid, i16, -1)

            def body8(cc, _):
                for u in range(8):
                    c = cc * 8 + u
                    csplat = jnp.zeros((_L,), jnp.int32) + c
                    v = plsc.load_gather(src_v, [csplat, col], mask=valid)
                    plsc.store_scatter(sw_v, [row16, csplat], v * sv)
                return 0

            lax.fori_loop(0, 8, body8, 0)

            @pl.when(parity == 0)
            def _():
                sia_v[pl.ds(slot * _L, _L)] = siv

            @pl.when(parity == 1)
            def _():
                sib_v[pl.ds(slot * _L, _L)] = siv

            @pl.when(slot == 3)
            def _():
                flush_fire(parity)

            cnts[65] = gc + 1

        def slow_pick(src_v, col_base, s_lo, s_hi):
            def scan_vreg(q, _):
                hv = mh_v[pl.ds(q * _L, _L)]
                iv = mi_v[pl.ds(q * _L, _L)]
                m = (hv >= s_lo) & (hv < s_hi) & (q * _L + iota < cnt)
                pos2 = plsc.cumsum(m.astype(jnp.int32)) - 1
                plsc.store_scatter(sth_v, [pos2], hv, mask=m)
                plsc.store_scatter(sti_v, [pos2], iv, mask=m)
                c2 = plsc.all_reduce_population_count(m)[0]

                @pl.when(c2 > 0)
                def _():
                    emit_group(src_v, col_base, sth_v[...], sti_v[...], c2)
                return 0

            lax.fori_loop(0, n_mv, scan_vreg, 0)

        def pick_slab(k, src_v, col_base, s_lo, s_hi):
            n_k = cnts[k]

            @pl.when((n_k > 0) & (n_k <= _BCAP))
            def _():
                def group(g, _):
                    @pl.when(n_k > g * _L)
                    def _():
                        h16 = bh_v[pl.ds(k * _BCAP + g * _L, _L)]
                        i16 = bi_v[pl.ds(k * _BCAP + g * _L, _L)]
                        emit_group(src_v, col_base, h16, i16,
                                   n_k - g * _L)
                    return 0

                lax.fori_loop(0, _BCAP // _L, group, 0)

            @pl.when(n_k > _BCAP)
            def _():
                slow_pick(src_v, col_base, s_lo, s_hi)

        def slab_pair(kk, _):
            k0 = 2 * kk
            wait_slab(slab0_v, sem_s0)
            off0 = slab_off(k0)
            pick_slab(k0, slab0_v, off0, jnp.maximum(lo, off0),
                      jnp.minimum(hi_dma, off0 + _SLAB))

            @pl.when(k0 + 2 < _N_SLABS)
            def _():
                fire_slab(k0 + 2, slab0_v, sem_s0)

            k1 = k0 + 1
            wait_slab(slab1_v, sem_s1)
            off1 = slab_off(k1)
            pick_slab(k1, slab1_v, off1, jnp.maximum(lo, off1),
                      jnp.minimum(hi_dma, off1 + _SLAB))

            @pl.when(k1 + 2 < _N_SLABS)
            def _():
                fire_slab(k1 + 2, slab1_v, sem_s1)

            return 0

        lax.fori_loop(0, _N_SLABS // 2, slab_pair, 0)

        pick_slab(jnp.int32(64), tail_v, jnp.int32(vtail),
                  jnp.int32(vtail), hi)

        gc = cnts[65]
        q = gc // 4
        r = gc % 4

        @pl.when(q >= 1)
        def _():
            flush_wait((q - 1) % 2)

        @pl.when((r == 0) & (q >= 2))
        def _():
            flush_wait(q % 2)

        flush_fire(jnp.int32(0))
        flush_fire(jnp.int32(1))
        flush_wait(jnp.int32(0))
        flush_wait(jnp.int32(1))

    return sc_kernel


def kernel(x, embed, scale):
    B, S = x.shape
    V, D = embed.shape
    N = B * S
    vtail = (V // 128) * 128

    t = x.astype(jnp.int32)
    prev = jnp.concatenate([jnp.zeros_like(t[:, :1]), t[:, :-1]], axis=1)
    scale_vec = jnp.full((_L,), scale, jnp.float32)
    embed_t = embed.T
    tail = embed[vtail:, :].T

    sc = _make_sc_kernel(N, V, D)
    out = sc(t.reshape(N), prev.reshape(N), scale_vec, tail, embed_t)
    return out[:, :D].reshape(B, S, D)


# speedup vs baseline: 1.0439x; 1.0439x over previous
import functools

import jax
import jax.numpy as jnp
from jax import lax
from jax.experimental import pallas as pl
from jax.experimental.pallas import tpu as pltpu
from jax.experimental.pallas import tpu_sc as plsc

_NC = 2
_NS = 16
_L = 16
_NW = _NC * _NS

_MULT_CUR = 36313
_MULT_PREV = 27191

_SLAB = 512
_N_SLABS = 64
_BCAP = 32
_GRP = 1024
_CHUNK_T = 1024


def _make_sc_kernel(N, V, D):
    vtail = (V // 128) * 128
    max_off = ((V - _SLAB) // _SLAB) * _SLAB
    per_tile = V // _NW
    mesh = plsc.VectorSubcoreMesh(core_axis_name="c", subcore_axis_name="s")

    @functools.partial(
        pl.kernel,
        out_type=jax.ShapeDtypeStruct((N, 2 * D), jnp.float32),
        mesh=mesh,
        scratch_types=[
            pltpu.VMEM((_CHUNK_T,), jnp.int32),
            pltpu.VMEM((_CHUNK_T,), jnp.int32),
            pltpu.VMEM((N,), jnp.int32),
            pltpu.VMEM((N,), jnp.int32),
            pltpu.VMEM((65 * _BCAP,), jnp.int32),
            pltpu.VMEM((65 * _BCAP,), jnp.int32),
            pltpu.VMEM((64, _SLAB), jnp.float32),
            pltpu.VMEM((64, _SLAB), jnp.float32),
            pltpu.VMEM((64, 64), jnp.float32),
            pltpu.VMEM((128, 128), jnp.float32),
            pltpu.VMEM((64,), jnp.int32),
            pltpu.VMEM((64,), jnp.int32),
            pltpu.VMEM((_L,), jnp.int32),
            pltpu.VMEM((_L,), jnp.int32),
            pltpu.VMEM((_L,), jnp.float32),
            pltpu.SMEM((66,), jnp.int32),
            pltpu.SemaphoreType.DMA,
            pltpu.SemaphoreType.DMA,
            pltpu.SemaphoreType.DMA,
            pltpu.SemaphoreType.DMA,
        ],
        compiler_params=pltpu.CompilerParams(needs_layout_passes=False),
    )
    def sc_kernel(t_hbm, p_hbm, s_hbm, tail_hbm, xt_hbm, out_hbm,
                  t_v, p_v, mh_v, mi_v, bh_v, bi_v, slab0_v, slab1_v,
                  tail_v, sw_v, sia_v, sib_v, sth_v, sti_v, s_v, cnts,
                  sem_in, sem_s0, sem_s1, sem_sc):
        wid = lax.axis_index("s") * _NC + lax.axis_index("c")
        lo = wid * per_tile
        hi = lo + per_tile
        hi_dma = jnp.minimum(hi, vtail)
        lo_al = (lo // _SLAB) * _SLAB
        iota = lax.iota(jnp.int32, _L)
        lane0 = iota == 0

        def slab_off(k):
            return pl.multiple_of(
                jnp.minimum(lo_al + k * _SLAB, max_off), _SLAB)

        def fire_slab(k, buf, sem):
            pltpu.async_copy(
                xt_hbm.at[:, pl.ds(slab_off(k), _SLAB)], buf, sem)

        def wait_slab(buf, sem):
            pltpu.make_async_copy(
                xt_hbm.at[:, pl.ds(0, _SLAB)], buf, sem).wait()

        fire_slab(0, slab0_v, sem_s0)
        fire_slab(1, slab1_v, sem_s1)
        pltpu.sync_copy(s_hbm, s_v)
        pltpu.async_copy(tail_hbm, tail_v, sem_in).wait()
        sv = s_v[...]

        for j in range(66):
            cnts[j] = jnp.int32(0)
        neg1 = jnp.full((_L,), -1, jnp.int32)
        for j in range(64 // _L):
            sia_v[pl.ds(j * _L, _L)] = neg1
            sib_v[pl.ds(j * _L, _L)] = neg1

        cnt = jnp.int32(0)
        for cc in range(N // _CHUNK_T):
            cp_t = pltpu.async_copy(
                t_hbm.at[pl.ds(cc * _CHUNK_T, _CHUNK_T)], t_v, sem_in)
            cp_p = pltpu.async_copy(
                p_hbm.at[pl.ds(cc * _CHUNK_T, _CHUNK_T)], p_v, sem_in)
            cp_t.wait()
            cp_p.wait()

            def hash_vreg(j, cnt, cc=cc):
                cur = t_v[pl.ds(j * _L, _L)]
                prv = p_v[pl.ds(j * _L, _L)]
                h = lax.bitwise_xor(
                    cur * _MULT_CUR, prv * _MULT_PREV) % (V - 1)
                m = (h >= lo) & (h < hi)
                pos = cnt + plsc.cumsum(m.astype(jnp.int32)) - 1
                plsc.store_scatter(mh_v, [pos], h, mask=m)
                plsc.store_scatter(
                    mi_v, [pos], cc * _CHUNK_T + j * _L + iota, mask=m)
                return cnt + plsc.all_reduce_population_count(m)[0]

            cnt = lax.fori_loop(0, _CHUNK_T // _L, hash_vreg, cnt)

        n_mv = (cnt + _L - 1) // _L

        def bucket_vreg(q, _):
            @pl.when(q * _L < cnt)
            def _():
                hv = mh_v[pl.ds(q * _L, _L)]
                iv = mi_v[pl.ds(q * _L, _L)]
                kv = jnp.where(hv >= vtail, 64, (hv - lo_al) >> 9)
                for ln in range(_L):
                    @pl.when(q * _L + ln < cnt)
                    def _(ln=ln):
                        k_s = kv[ln]
                        c_s = cnts[k_s]
                        cnts[k_s] = c_s + 1

                        @pl.when(c_s < _BCAP)
                        def _():
                            p16 = jnp.zeros((_L,), jnp.int32) + (
                                k_s * _BCAP + c_s)
                            plsc.store_scatter(
                                bh_v, [p16],
                                jnp.zeros((_L,), jnp.int32) + hv[ln],
                                mask=lane0)
                            plsc.store_scatter(
                                bi_v, [p16],
                                jnp.zeros((_L,), jnp.int32) + iv[ln],
                                mask=lane0)
            return 0

        lax.fori_loop(0, N // _L, bucket_vreg, 0)

        def flush_wait(parity):
            @pl.when(parity == 0)
            def _():
                pltpu.make_async_copy(
                    sw_v.at[pl.ds(0, 64), :],
                    out_hbm.at[plsc.Indices(sia_v, ignored_value=-1)],
                    sem_sc).wait()

            @pl.when(parity == 1)
            def _():
                pltpu.make_async_copy(
                    sw_v.at[pl.ds(64, 64), :],
                    out_hbm.at[plsc.Indices(sib_v, ignored_value=-1)],
                    sem_sc).wait()

        def flush_fire(parity):
            @pl.when(parity == 0)
            def _():
                pltpu.async_copy(
                    sw_v.at[pl.ds(0, 64), :],
                    out_hbm.at[plsc.Indices(sia_v, ignored_value=-1)],
                    sem_sc)

            @pl.when(parity == 1)
            def _():
                pltpu.async_copy(
                    sw_v.at[pl.ds(64, 64), :],
                    out_hbm.at[plsc.Indices(sib_v, ignored_value=-1)],
                    sem_sc)

        def emit_group(src_v, col_base, h16, i16, nvalid):
            gc = cnts[65]
            slot = gc % 4
            parity = (gc // 4) % 2

            @pl.when((slot == 0) & (gc >= 8))
            def _():
                flush_wait(parity)

            valid = iota < nvalid
            col = h16 - col_base
            row16 = parity * 64 + slot * _L + iota
            siv = jnp.where(valid, i16, -1)

            def body8(cc, _):
                for u in range(8):
                    c = cc * 8 + u
                    csplat = jnp.zeros((_L,), jnp.int32) + c
                    v = plsc.load_gather(src_v, [csplat, col], mask=valid)
                    plsc.store_scatter(sw_v, [row16, csplat], v * sv)
                return 0

            lax.fori_loop(0, 8, body8, 0)

            @pl.when(parity == 0)
            def _():
                sia_v[pl.ds(slot * _L, _L)] = siv

            @pl.when(parity == 1)
            def _():
                sib_v[pl.ds(slot * _L, _L)] = siv

            @pl.when(slot == 3)
            def _():
                flush_fire(parity)

            cnts[65] = gc + 1

        def slow_pick(src_v, col_base, s_lo, s_hi):
            def scan_vreg(q, _):
                hv = mh_v[pl.ds(q * _L, _L)]
                iv = mi_v[pl.ds(q * _L, _L)]
                m = (hv >= s_lo) & (hv < s_hi) & (q * _L + iota < cnt)
                pos2 = plsc.cumsum(m.astype(jnp.int32)) - 1
                plsc.store_scatter(sth_v, [pos2], hv, mask=m)
                plsc.store_scatter(sti_v, [pos2], iv, mask=m)
                c2 = plsc.all_reduce_population_count(m)[0]

                @pl.when(c2 > 0)
                def _():
                    emit_group(src_v, col_base, sth_v[...], sti_v[...], c2)
                return 0

            lax.fori_loop(0, n_mv, scan_vreg, 0)

        def pick_slab(k, src_v, col_base, s_lo, s_hi):
            n_k = cnts[k]

            @pl.when((n_k > 0) & (n_k <= _BCAP))
            def _():
                def group(g, _):
                    @pl.when(n_k > g * _L)
                    def _():
                        h16 = bh_v[pl.ds(k * _BCAP + g * _L, _L)]
                        i16 = bi_v[pl.ds(k * _BCAP + g * _L, _L)]
                        emit_group(src_v, col_base, h16, i16,
                                   n_k - g * _L)
                    return 0

                lax.fori_loop(0, _BCAP // _L, group, 0)

            @pl.when(n_k > _BCAP)
            def _():
                slow_pick(src_v, col_base, s_lo, s_hi)

        def slab_pair(kk, _):
            k0 = 2 * kk
            wait_slab(slab0_v, sem_s0)
            off0 = slab_off(k0)
            pick_slab(k0, slab0_v, off0, jnp.maximum(lo, off0),
                      jnp.minimum(hi_dma, off0 + _SLAB))

            @pl.when(k0 + 2 < _N_SLABS)
            def _():
                fire_slab(k0 + 2, slab0_v, sem_s0)

            k1 = k0 + 1
            wait_slab(slab1_v, sem_s1)
            off1 = slab_off(k1)
            pick_slab(k1, slab1_v, off1, jnp.maximum(lo, off1),
                      jnp.minimum(hi_dma, off1 + _SLAB))

            @pl.when(k1 + 2 < _N_SLABS)
            def _():
                fire_slab(k1 + 2, slab1_v, sem_s1)

            return 0

        lax.fori_loop(0, _N_SLABS // 2, slab_pair, 0)

        pick_slab(jnp.int32(64), tail_v, jnp.int32(vtail),
                  jnp.int32(vtail), hi)

        gc = cnts[65]
        q = gc // 4
        r = gc % 4

        @pl.when(q >= 1)
        def _():
            flush_wait((q - 1) % 2)

        @pl.when((r == 0) & (q >= 2))
        def _():
            flush_wait(q % 2)

        flush_fire(jnp.int32(0))
        flush_fire(jnp.int32(1))
        flush_wait(jnp.int32(0))
        flush_wait(jnp.int32(1))

    return sc_kernel


def kernel(x, embed, scale):
    B, S = x.shape
    V, D = embed.shape
    N = B * S
    vtail = (V // 128) * 128

    t = x.astype(jnp.int32)
    prev = jnp.concatenate([jnp.zeros_like(t[:, :1]), t[:, :-1]], axis=1)
    scale_vec = jnp.full((_L,), scale, jnp.float32)
    embed_t = embed.T
    tail = embed[vtail:, :].T

    sc = _make_sc_kernel(N, V, D)
    out = sc(t.reshape(N), prev.reshape(N), scale_vec, tail, embed_t)
    return out[:, :D].reshape(B, S, D)
